# baseline (speedup 1.0000x reference)
import jax
import jax.numpy as jnp
from jax import lax
from jax.experimental import pallas as pl
from jax.experimental.pallas import tpu as pltpu
from jax.experimental.pallas import tpu_sc as plsc

B = 4096
L = 50
D = 32
NC = 2
NS = 16
NW = NC * NS
ROWS_PW = B // NW
CB = 16
NCHUNK = ROWS_PW // CB
N = CB * L
SUB = 80
NSUB = N // SUB
LANES = 16


def _body(words_hbm, hp_hbm, hn_hbm, mask_hbm, v_hbm, w_hbm, vb_hbm, wb_hbm,
          pos_out, neg_out,
          words_v, hp_v, hn_v, mask_v,
          idxw, idxhp, idxhn, maskf,
          wr, vpr, vnr, wbv, vbp, vbn,
          scp, scn, outp_v, outn_v, sem):
    cid = lax.axis_index("c")
    sid = lax.axis_index("s")
    wid = sid * NC + cid
    iota = lax.iota(jnp.int32, LANES)

    @pl.loop(0, NCHUNK)
    def _chunk(c):
        pos0 = wid * (ROWS_PW * L) + c * N
        row0 = wid * ROWS_PW + c * CB

        pltpu.sync_copy(words_hbm.at[pl.ds(pos0, N)], words_v)
        pltpu.sync_copy(hp_hbm.at[pl.ds(pos0, N)], hp_v)
        pltpu.sync_copy(hn_hbm.at[pl.ds(pos0, N)], hn_v)
        pltpu.sync_copy(mask_hbm.at[pl.ds(pos0, N)], mask_v)

        @pl.loop(0, N // LANES)
        def _pha(g):
            sl = pl.ds(g * LANES, LANES)
            p = g * LANES + iota
            m = mask_v[sl] != 0
            root = m | (p % L == 0)
            idxw[sl] = jnp.where(m, 0, words_v[sl])
            maskf[sl] = jnp.where(root, 0.0, 1.0)

        @pl.loop(0, N // LANES)
        def _pha2(g):
            sl = pl.ds(g * LANES, LANES)
            p = g * LANES + iota
            l = p % L
            rowbase = p - l
            root = (mask_v[sl] != 0) | (l == 0)
            hp = jnp.where(root, 0, hp_v[sl])
            hn = jnp.where(root, 0, hn_v[sl])
            idxhp[sl] = plsc.load_gather(idxw, [rowbase + hp])
            idxhn[sl] = plsc.load_gather(idxw, [rowbase + hn])

        pltpu.async_copy(w_hbm.at[idxw], wr, sem)
        pltpu.async_copy(v_hbm.at[idxhp], vpr, sem)
        pltpu.async_copy(v_hbm.at[idxhn], vnr, sem)
        pltpu.async_copy(wb_hbm.at[idxw], wbv, sem)
        pltpu.async_copy(vb_hbm.at[idxhp], vbp, sem)
        pltpu.async_copy(vb_hbm.at[idxhn], vbn, sem)
        pltpu.make_async_copy(w_hbm.at[idxw], wr, sem).wait()
        pltpu.make_async_copy(v_hbm.at[idxhp], vpr, sem).wait()
        pltpu.make_async_copy(v_hbm.at[idxhn], vnr, sem).wait()
        pltpu.make_async_copy(wb_hbm.at[idxw], wbv, sem).wait()
        pltpu.make_async_copy(vb_hbm.at[idxhp], vbp, sem).wait()
        pltpu.make_async_copy(vb_hbm.at[idxhn], vbn, sem).wait()

        @pl.loop(0, N // LANES)
        def _phc(g):
            sl = pl.ds(g * LANES, LANES)
            p16 = g * LANES + iota
            accp = jnp.zeros((LANES,), jnp.float32)
            accn = jnp.zeros((LANES,), jnp.float32)
            for d in range(D):
                dv = jnp.full((LANES,), d, jnp.int32)
                wv = plsc.load_gather(wr, [p16, dv])
                accp = accp + wv * plsc.load_gather(vpr, [p16, dv])
                accn = accn + wv * plsc.load_gather(vnr, [p16, dv])
            m = maskf[sl]
            scp[sl] = (accp + vbp[sl] + wbv[sl]) * m
            scn[sl] = (accn + vbn[sl] + wbv[sl]) * m

        accp = jnp.zeros((LANES,), jnp.float32)
        accn = jnp.zeros((LANES,), jnp.float32)
        rbase = iota * L
        for l in range(L):
            accp = accp + plsc.load_gather(scp, [rbase + l])
            accn = accn + plsc.load_gather(scn, [rbase + l])
        outp_v[...] = accp
        outn_v[...] = accn
        pltpu.sync_copy(outp_v, pos_out.at[pl.ds(row0, CB)])
        pltpu.sync_copy(outn_v, neg_out.at[pl.ds(row0, CB)])


def kernel(batch_id, words, head_ids, negative_head_ids, mask, V, W, vb, wb):
    del batch_id
    words_f = words.reshape(-1).astype(jnp.int32)
    hp_f = head_ids.reshape(-1).astype(jnp.int32)
    hn_f = negative_head_ids.reshape(-1).astype(jnp.int32)
    mask_f = mask.reshape(-1).astype(jnp.int32)

    mesh = plsc.VectorSubcoreMesh(core_axis_name="c", subcore_axis_name="s")
    f = pl.kernel(
        _body,
        out_type=(
            jax.ShapeDtypeStruct((B,), jnp.float32),
            jax.ShapeDtypeStruct((B,), jnp.float32),
        ),
        mesh=mesh,
        compiler_params=pltpu.CompilerParams(needs_layout_passes=False,
                                             use_tc_tiling_on_sc=False),
        scratch_types=[
            pltpu.VMEM((N,), jnp.int32),
            pltpu.VMEM((N,), jnp.int32),
            pltpu.VMEM((N,), jnp.int32),
            pltpu.VMEM((N,), jnp.int32),
            pltpu.VMEM((N,), jnp.int32),
            pltpu.VMEM((N,), jnp.int32),
            pltpu.VMEM((N,), jnp.int32),
            pltpu.VMEM((N,), jnp.float32),
            pltpu.VMEM((N, D), jnp.float32),
            pltpu.VMEM((N, D), jnp.float32),
            pltpu.VMEM((N, D), jnp.float32),
            pltpu.VMEM((N,), jnp.float32),
            pltpu.VMEM((N,), jnp.float32),
            pltpu.VMEM((N,), jnp.float32),
            pltpu.VMEM((N,), jnp.float32),
            pltpu.VMEM((N,), jnp.float32),
            pltpu.VMEM((LANES,), jnp.float32),
            pltpu.VMEM((LANES,), jnp.float32),
            pltpu.SemaphoreType.DMA,
        ],
    )
    return f(words_f, hp_f, hn_f, mask_f, V, W, vb, wb)


# speedup vs baseline: 1.7648x; 1.7648x over previous
import jax
import jax.numpy as jnp
from jax import lax
from jax.experimental import pallas as pl
from jax.experimental.pallas import tpu as pltpu
from jax.experimental.pallas import tpu_sc as plsc

B = 4096
L = 50
D = 32
NC = 2
NS = 16
NW = NC * NS
ROWS_PW = B // NW
CB = 16
NCHUNK = ROWS_PW // CB
N = CB * L
LANES = 16


def _body(words_hbm, hp_hbm, hn_hbm, mask_hbm, v_hbm, w_hbm, vb_hbm, wb_hbm,
          pos_out, neg_out,
          words_v, hp_v, hn_v, mask_v,
          idxw, qp_v, qn_v, maskf,
          wr, vr, wbv, vbv,
          scp, scn, outp_v, outn_v, sem):
    cid = lax.axis_index("c")
    sid = lax.axis_index("s")
    wid = sid * NC + cid
    iota = lax.iota(jnp.int32, LANES)

    @pl.loop(0, NCHUNK)
    def _chunk(c):
        pos0 = wid * (ROWS_PW * L) + c * N
        row0 = wid * ROWS_PW + c * CB

        pltpu.sync_copy(words_hbm.at[pl.ds(pos0, N)], words_v)
        pltpu.sync_copy(hp_hbm.at[pl.ds(pos0, N)], hp_v)
        pltpu.sync_copy(hn_hbm.at[pl.ds(pos0, N)], hn_v)
        pltpu.sync_copy(mask_hbm.at[pl.ds(pos0, N)], mask_v)

        @pl.loop(0, N // LANES)
        def _pha(g):
            sl = pl.ds(g * LANES, LANES)
            p = g * LANES + iota
            l = p % L
            rowbase = p - l
            m = mask_v[sl] != 0
            root = m | (l == 0)
            idxw[sl] = jnp.where(m, 0, words_v[sl])
            qp_v[sl] = rowbase + jnp.where(root, 0, hp_v[sl])
            qn_v[sl] = rowbase + jnp.where(root, 0, hn_v[sl])
            maskf[sl] = jnp.where(root, 0.0, 1.0)

        pltpu.async_copy(w_hbm.at[idxw], wr, sem)
        pltpu.async_copy(v_hbm.at[idxw], vr, sem)
        pltpu.async_copy(wb_hbm.at[idxw], wbv, sem)
        pltpu.async_copy(vb_hbm.at[idxw], vbv, sem)
        pltpu.make_async_copy(w_hbm.at[idxw], wr, sem).wait()
        pltpu.make_async_copy(v_hbm.at[idxw], vr, sem).wait()
        pltpu.make_async_copy(wb_hbm.at[idxw], wbv, sem).wait()
        pltpu.make_async_copy(vb_hbm.at[idxw], vbv, sem).wait()

        @pl.loop(0, N // LANES)
        def _phc(g):
            sl = pl.ds(g * LANES, LANES)
            p16 = g * LANES + iota
            qp16 = qp_v[sl]
            qn16 = qn_v[sl]
            accp = jnp.zeros((LANES,), jnp.float32)
            accn = jnp.zeros((LANES,), jnp.float32)
            for d in range(D):
                dv = jnp.full((LANES,), d, jnp.int32)
                wv = plsc.load_gather(wr, [p16, dv])
                accp = accp + wv * plsc.load_gather(vr, [qp16, dv])
                accn = accn + wv * plsc.load_gather(vr, [qn16, dv])
            m = maskf[sl]
            wb16 = wbv[sl]
            scp[sl] = (accp + plsc.load_gather(vbv, [qp16]) + wb16) * m
            scn[sl] = (accn + plsc.load_gather(vbv, [qn16]) + wb16) * m

        accp = jnp.zeros((LANES,), jnp.float32)
        accn = jnp.zeros((LANES,), jnp.float32)
        rbase = iota * L
        for l in range(L):
            accp = accp + plsc.load_gather(scp, [rbase + l])
            accn = accn + plsc.load_gather(scn, [rbase + l])
        outp_v[...] = accp
        outn_v[...] = accn
        pltpu.sync_copy(outp_v, pos_out.at[pl.ds(row0, CB)])
        pltpu.sync_copy(outn_v, neg_out.at[pl.ds(row0, CB)])


def kernel(batch_id, words, head_ids, negative_head_ids, mask, V, W, vb, wb):
    del batch_id
    words_f = words.reshape(-1).astype(jnp.int32)
    hp_f = head_ids.reshape(-1).astype(jnp.int32)
    hn_f = negative_head_ids.reshape(-1).astype(jnp.int32)
    mask_f = mask.reshape(-1).astype(jnp.int32)

    mesh = plsc.VectorSubcoreMesh(core_axis_name="c", subcore_axis_name="s")
    f = pl.kernel(
        _body,
        out_type=(
            jax.ShapeDtypeStruct((B,), jnp.float32),
            jax.ShapeDtypeStruct((B,), jnp.float32),
        ),
        mesh=mesh,
        compiler_params=pltpu.CompilerParams(needs_layout_passes=False,
                                             use_tc_tiling_on_sc=False),
        scratch_types=[
            pltpu.VMEM((N,), jnp.int32),
            pltpu.VMEM((N,), jnp.int32),
            pltpu.VMEM((N,), jnp.int32),
            pltpu.VMEM((N,), jnp.int32),
            pltpu.VMEM((N,), jnp.int32),
            pltpu.VMEM((N,), jnp.int32),
            pltpu.VMEM((N,), jnp.int32),
            pltpu.VMEM((N,), jnp.float32),
            pltpu.VMEM((N, D), jnp.float32),
            pltpu.VMEM((N, D), jnp.float32),
            pltpu.VMEM((N,), jnp.float32),
            pltpu.VMEM((N,), jnp.float32),
            pltpu.VMEM((N,), jnp.float32),
            pltpu.VMEM((N,), jnp.float32),
            pltpu.VMEM((LANES,), jnp.float32),
            pltpu.VMEM((LANES,), jnp.float32),
            pltpu.SemaphoreType.DMA,
        ],
    )
    return f(words_f, hp_f, hn_f, mask_f, V, W, vb, wb)
